# baseline (speedup 1.0000x reference)
import jax
import jax.numpy as jnp
from jax import lax
from jax.experimental import pallas as pl
from jax.experimental.pallas import tpu as pltpu
from jax.experimental.pallas import tpu_sc as plsc

VOCAB = 100000
D = 60
DP = 64
P = 62
B, L = 4096, 200
N = B * L

NC, NS = 2, 16
NW = NC * NS
BPW = B // NW
CB = L
GS = ((0, 128), (128, 72))
NCHUNK = BPW
NIC = 13


def _emb_body(word_h, head_h, tail_h, wt_h, comb_h, out_h,
              iw, ih, it, ic, w_buf, c_buf, o_buf,
              sem_i, sem_w, sem_c, sem_o):
    core = lax.axis_index("c")
    sub = lax.axis_index("s")
    wid = sub * NC + core
    base = wid * BPW * CB

    def load_idx(off, nb):
        pltpu.async_copy(word_h.at[pl.ds(off, CB)], iw.at[nb], sem_i)
        pltpu.async_copy(head_h.at[pl.ds(off, CB)], ih.at[nb], sem_i)
        pltpu.async_copy(tail_h.at[pl.ds(off, CB)], it.at[nb], sem_i)

    def wait_idx(nb):
        for r in (iw, ih, it):
            pltpu.make_async_copy(word_h.at[pl.ds(0, CB)], r.at[nb], sem_i).wait()

    def compute_ic(nb):
        @plsc.parallel_loop(0, NIC, unroll=4)
        def _(i):
            j = jnp.minimum(i * 16, CB - 16)
            w = iw[nb, pl.ds(j, 16)]
            h = ih[nb, pl.ds(j, 16)]
            t = it[nb, pl.ds(j, 16)]
            ic[nb, pl.ds(j, 16)] = h * P + t + jnp.where(w == 0, P * P, 0)

    def fire_gathers(nb):
        for (o, g) in GS:
            pltpu.async_copy(wt_h.at[iw.at[nb, pl.ds(o, g)]],
                             w_buf.at[nb, pl.ds(o, g)], sem_w)
            pltpu.async_copy(comb_h.at[ic.at[nb, pl.ds(o, g)]],
                             c_buf.at[nb, pl.ds(o, g)], sem_c)

    def wait_gathers(nb):
        for (o, g) in GS:
            pltpu.make_async_copy(wt_h.at[iw.at[nb, pl.ds(o, g)]],
                                  w_buf.at[nb, pl.ds(o, g)], sem_w).wait()
            pltpu.make_async_copy(comb_h.at[ic.at[nb, pl.ds(o, g)]],
                                  c_buf.at[nb, pl.ds(o, g)], sem_c).wait()

    def drain_store():
        pltpu.make_async_copy(o_buf.at[0], out_h.at[0], sem_o).wait()

    load_idx(base, 0)
    wait_idx(0)
    compute_ic(0)
    fire_gathers(0)
    load_idx(base + CB, 1)

    @pl.loop(0, NCHUNK, step=2)
    def _(g0):
        for b in range(2):
            nb = 1 - b
            g = g0 + b
            off = base + g * CB

            wait_gathers(b)

            @pl.when(g < NCHUNK - 1)
            def _():
                wait_idx(nb)
                compute_ic(nb)
                fire_gathers(nb)

            @pl.when(g < NCHUNK - 2)
            def _():
                load_idx(off + 2 * CB, b)

            @pl.when(g >= 2)
            def _():
                drain_store()

            @plsc.parallel_loop(0, CB, unroll=4)
            def _(r):
                o_buf[b, r, pl.ds(0, 16)] = (
                    w_buf[b, r, pl.ds(0, 16)] + c_buf[b, r, pl.ds(0, 16)])
                o_buf[b, r, pl.ds(16, 16)] = (
                    w_buf[b, r, pl.ds(16, 16)] + c_buf[b, r, pl.ds(16, 16)])
                o_buf[b, r, pl.ds(32, 16)] = (
                    w_buf[b, r, pl.ds(32, 16)] + c_buf[b, r, pl.ds(32, 16)])
                o_buf[b, r, pl.ds(44, 16)] = (
                    w_buf[b, r, pl.ds(44, 16)] + c_buf[b, r, pl.ds(44, 16)])

            pltpu.async_copy(o_buf.at[b], out_h.at[wid * BPW + g], sem_o)

    drain_store()
    drain_store()


@jax.jit
def _emb(word, head, tail, wt, comb):
    mesh = plsc.VectorSubcoreMesh(core_axis_name="c", subcore_axis_name="s")
    f = pl.kernel(
        _emb_body,
        mesh=mesh,
        compiler_params=pltpu.CompilerParams(use_tc_tiling_on_sc=False),
        out_type=jax.ShapeDtypeStruct((B, L, D), jnp.float32),
        scratch_types=[
            pltpu.VMEM((2, CB), jnp.int32),
            pltpu.VMEM((2, CB), jnp.int32),
            pltpu.VMEM((2, CB), jnp.int32),
            pltpu.VMEM((2, CB), jnp.int32),
            pltpu.VMEM((2, CB, DP), jnp.float32),
            pltpu.VMEM((2, CB, DP), jnp.float32),
            pltpu.VMEM((2, CB, D), jnp.float32),
            pltpu.SemaphoreType.DMA,
            pltpu.SemaphoreType.DMA,
            pltpu.SemaphoreType.DMA,
            pltpu.SemaphoreType.DMA,
        ],
    )
    return f(word, head, tail, wt, comb)


def kernel(word, head, tail, word_table, head_table, tail_table):
    ht0 = head_table.at[0].set(0.0)
    tt0 = tail_table.at[0].set(0.0)
    base = ht0[:, None, :] + tt0[None, :, :]
    base = base.reshape(P * P, D)
    comb = jnp.concatenate([base, base - word_table[0]], axis=0)
    comb = jnp.pad(comb, ((0, 0), (0, DP - D)))
    wt = jnp.pad(word_table, ((0, 0), (0, DP - D)))

    return _emb(
        word.reshape(-1).astype(jnp.int32),
        head.reshape(-1).astype(jnp.int32),
        tail.reshape(-1).astype(jnp.int32),
        wt,
        comb,
    )


# speedup vs baseline: 1.1588x; 1.1588x over previous
import jax
import jax.numpy as jnp
from jax import lax
from jax.experimental import pallas as pl
from jax.experimental.pallas import tpu as pltpu
from jax.experimental.pallas import tpu_sc as plsc

VOCAB = 100000
D = 60
DP = 128
P = 62
B, L = 4096, 200
N = B * L

NC, NS = 2, 16
NW = NC * NS
TPW = N // NW
CB = 128
G = 128
NG = CB // G
NCHUNK = TPW // CB


def _emb_body(word_h, head_h, tail_h, wt_h, comb_h, out_h,
              iw, ih, it, ic, w_buf, c_buf, o_buf,
              sem_i, sem_w, sem_c, sem_o):
    core = lax.axis_index("c")
    sub = lax.axis_index("s")
    wid = sub * NC + core
    base = wid * TPW

    def load_idx(off, nb):
        pltpu.async_copy(word_h.at[pl.ds(off, CB)], iw.at[nb], sem_i)
        pltpu.async_copy(head_h.at[pl.ds(off, CB)], ih.at[nb], sem_i)
        pltpu.async_copy(tail_h.at[pl.ds(off, CB)], it.at[nb], sem_i)

    def wait_idx(nb):
        for r in (iw, ih, it):
            pltpu.make_async_copy(word_h.at[pl.ds(0, CB)], r.at[nb], sem_i).wait()

    def compute_ic(nb):
        @plsc.parallel_loop(0, CB // 16, unroll=4)
        def _(i):
            j = i * 16
            w = iw[nb, pl.ds(j, 16)]
            h = ih[nb, pl.ds(j, 16)]
            t = it[nb, pl.ds(j, 16)]
            ic[nb, pl.ds(j, 16)] = h * P + t + jnp.where(w == 0, P * P, 0)

    def fire_gathers(nb):
        for j in range(NG):
            pltpu.async_copy(wt_h.at[iw.at[nb, pl.ds(j * G, G)]],
                             w_buf.at[nb, pl.ds(j * G, G)], sem_w)
            pltpu.async_copy(comb_h.at[ic.at[nb, pl.ds(j * G, G)]],
                             c_buf.at[nb, pl.ds(j * G, G)], sem_c)

    def wait_gathers(nb):
        for j in range(NG):
            pltpu.make_async_copy(wt_h.at[iw.at[nb, pl.ds(j * G, G)]],
                                  w_buf.at[nb, pl.ds(j * G, G)], sem_w).wait()
            pltpu.make_async_copy(comb_h.at[ic.at[nb, pl.ds(j * G, G)]],
                                  c_buf.at[nb, pl.ds(j * G, G)], sem_c).wait()

    def drain_store():
        pltpu.make_async_copy(o_buf.at[0], out_h.at[pl.ds(0, CB)], sem_o).wait()

    load_idx(base, 0)
    wait_idx(0)
    compute_ic(0)
    fire_gathers(0)
    load_idx(base + CB, 1)

    @pl.loop(0, NCHUNK, step=2)
    def _(g0):
        for b in range(2):
            nb = 1 - b
            g = g0 + b
            off = base + g * CB

            wait_gathers(b)

            @pl.when(g < NCHUNK - 1)
            def _():
                wait_idx(nb)
                compute_ic(nb)
                fire_gathers(nb)

            @pl.when(g < NCHUNK - 2)
            def _():
                load_idx(off + 2 * CB, b)

            @pl.when(g >= 2)
            def _():
                drain_store()

            @plsc.parallel_loop(0, CB, unroll=4)
            def _(r):
                o_buf[b, r, pl.ds(0, 16)] = (
                    w_buf[b, r, pl.ds(0, 16)] + c_buf[b, r, pl.ds(0, 16)])
                o_buf[b, r, pl.ds(16, 16)] = (
                    w_buf[b, r, pl.ds(16, 16)] + c_buf[b, r, pl.ds(16, 16)])
                o_buf[b, r, pl.ds(32, 16)] = (
                    w_buf[b, r, pl.ds(32, 16)] + c_buf[b, r, pl.ds(32, 16)])
                o_buf[b, r, pl.ds(44, 16)] = (
                    w_buf[b, r, pl.ds(44, 16)] + c_buf[b, r, pl.ds(44, 16)])

            pltpu.async_copy(o_buf.at[b], out_h.at[pl.ds(off, CB)], sem_o)

    drain_store()
    drain_store()


@jax.jit
def _emb(word, head, tail, wt, comb):
    mesh = plsc.VectorSubcoreMesh(core_axis_name="c", subcore_axis_name="s")
    f = pl.kernel(
        _emb_body,
        mesh=mesh,
        out_type=jax.ShapeDtypeStruct((N, D), jnp.float32),
        scratch_types=[
            pltpu.VMEM((2, CB), jnp.int32),
            pltpu.VMEM((2, CB), jnp.int32),
            pltpu.VMEM((2, CB), jnp.int32),
            pltpu.VMEM((2, CB), jnp.int32),
            pltpu.VMEM((2, CB, DP), jnp.float32),
            pltpu.VMEM((2, CB, DP), jnp.float32),
            pltpu.VMEM((2, CB, D), jnp.float32),
            pltpu.SemaphoreType.DMA,
            pltpu.SemaphoreType.DMA,
            pltpu.SemaphoreType.DMA,
            pltpu.SemaphoreType.DMA,
        ],
    )
    return f(word, head, tail, wt, comb)


def kernel(word, head, tail, word_table, head_table, tail_table):
    ht0 = head_table.at[0].set(0.0)
    tt0 = tail_table.at[0].set(0.0)
    base = ht0[:, None, :] + tt0[None, :, :]
    base = base.reshape(P * P, D)
    comb = jnp.concatenate([base, base - word_table[0]], axis=0)
    comb = jnp.pad(comb, ((0, 0), (0, DP - D)))
    wt = jnp.pad(word_table, ((0, 0), (0, DP - D)))

    return _emb(
        word.reshape(-1).astype(jnp.int32),
        head.reshape(-1).astype(jnp.int32),
        tail.reshape(-1).astype(jnp.int32),
        wt,
        comb,
    ).reshape(B, L, D)
